# baseline (speedup 1.0000x reference)
import jax
import jax.numpy as jnp
from jax import lax
from jax.experimental import pallas as pl
from jax.experimental.pallas import tpu as pltpu
from jax.experimental.pallas import tpu_sc as plsc

B, N, K, H = 8, 64, 128, 768
V = 100000
L = K + 1
CLS_SEG = 1
EPS = 1e-12

NC, NS = 2, 16
NW = NC * NS
CL = 16
WPB = N // CL
NSL = H // 16
PCH = 16


def _rsqrt(x):
    xi = lax.bitcast_convert_type(x, jnp.int32)
    yi = jnp.full((16,), 0x5F3759DF, jnp.int32) - (xi >> 1)
    y = lax.bitcast_convert_type(yi, jnp.float32)
    for _ in range(3):
        y = y * (1.5 - 0.5 * x * y * y)
    return y


def _unit_norm(src, dst, seg_buf, pos_buf, pj, gam_buf, bet_buf, stats):
    lanes = lax.iota(jnp.int32, 16)

    def j_body(j, acc):
        sl = pl.ds(pl.multiple_of(j * 16, 16), 16)
        pos_j = pos_buf[pj, sl]
        out = []
        for i in range(16):
            x = src[i, sl] + seg_buf[i, sl] + pos_j
            dst[i, sl] = x
            out.append(acc[2 * i] + x)
            out.append(acc[2 * i + 1] + x * x)
        return tuple(out)

    acc0 = tuple(jnp.zeros((16,), jnp.float32) for _ in range(32))
    acc = lax.fori_loop(0, NSL, j_body, acc0)
    for i in range(16):
        plsc.store_scatter(stats, [lanes + 17 * i], acc[2 * i])
        plsc.store_scatter(stats, [lanes + (272 + 17 * i)], acc[2 * i + 1])
    tot_s = jnp.zeros((16,), jnp.float32)
    tot_q = jnp.zeros((16,), jnp.float32)
    for c in range(16):
        tot_s = tot_s + plsc.load_gather(stats, [lanes * 17 + c])
        tot_q = tot_q + plsc.load_gather(stats, [lanes * 17 + (272 + c)])
    mean = tot_s * (1.0 / H)
    msq = tot_q * (1.0 / H)
    rinv = _rsqrt(msq - mean * mean + EPS)
    shift = -mean * rinv

    for h in (0, 8):
        ri = [jnp.full((16,), rinv[i], jnp.float32) for i in range(h, h + 8)]
        sh = [jnp.full((16,), shift[i], jnp.float32) for i in range(h, h + 8)]

        def j2_body(j, carry, h=h, ri=ri, sh=sh):
            sl = pl.ds(pl.multiple_of(j * 16, 16), 16)
            g = gam_buf[sl]
            b = bet_buf[sl]
            for i in range(h, h + 8):
                dst[i, sl] = (dst[i, sl] * ri[i - h] + sh[i - h]) * g + b
            return carry

        lax.fori_loop(0, NSL, j2_body, 0, unroll=2)


def _body(ids_ref, word_ref, pos_ref, seg_ref, gamma_ref, beta_ref,
          out_ref,
          ids_buf, gbuf0, gbuf1, obuf0, obuf1,
          pos_buf, seg_buf, gam_buf, bet_buf, stats,
          gsem0, gsem1, ssem0, ssem1):
    w = lax.axis_index("s") * NC + lax.axis_index("c")
    bb = w // WPB
    n0 = (w % WPB) * CL
    gbuf = (gbuf0, gbuf1)
    obuf = (obuf0, obuf1)
    gsem = (gsem0, gsem1)
    ssem = (ssem0, ssem1)

    pltpu.sync_copy(gamma_ref, gam_buf)
    pltpu.sync_copy(beta_ref, bet_buf)
    pltpu.sync_copy(seg_ref.at[pl.ds(n0, CL)], seg_buf)
    pltpu.sync_copy(ids_ref.at[pl.ds(bb * (L * N), L * N)], ids_buf)

    def _idx(u):
        return ids_buf.at[pl.ds(pl.multiple_of(u * N + n0, 8), CL)]

    for s in range(2):
        pltpu.async_copy(word_ref.at[_idx(s)], gbuf[s], gsem[s])
        pltpu.async_copy(seg_buf, out_ref.at[bb, s, pl.ds(n0, CL)], ssem[s])

    def step(k, carry):
        for s in range(2):
            u = k * 2 + s
            pltpu.make_async_copy(
                pos_ref.at[pl.ds(0, CL)], obuf[s], ssem[s]).wait()
            pltpu.make_async_copy(
                word_ref.at[_idx(u)], gbuf[s], gsem[s]).wait()

            if s == 0:
                @pl.when(lax.rem(u, PCH) == 0)
                def _():
                    lo = pl.multiple_of(u, PCH)
                    pltpu.sync_copy(pos_ref.at[pl.ds(lo, PCH)], pos_buf)

            _unit_norm(gbuf[s], obuf[s], seg_buf, pos_buf, lax.rem(u, PCH),
                       gam_buf, bet_buf, stats)

            @pl.when(u + 2 < L)
            def _():
                pltpu.async_copy(word_ref.at[_idx(u + 2)], gbuf[s], gsem[s])

            pltpu.async_copy(obuf[s], out_ref.at[bb, u, pl.ds(n0, CL)],
                             ssem[s])
        return carry

    lax.fori_loop(0, K // 2, step, 0)

    pltpu.make_async_copy(pos_ref.at[pl.ds(0, CL)], obuf0, ssem0).wait()
    pltpu.make_async_copy(word_ref.at[_idx(K)], gbuf0, gsem0).wait()
    pltpu.sync_copy(pos_ref.at[pl.ds(K, 8)], pos_buf.at[pl.ds(0, 8)])
    _unit_norm(gbuf0, obuf0, seg_buf, pos_buf, 0, gam_buf, bet_buf, stats)
    pltpu.sync_copy(obuf0, out_ref.at[bb, K, pl.ds(n0, CL)])
    pltpu.make_async_copy(pos_ref.at[pl.ds(0, CL)], obuf1, ssem1).wait()


_sc_call = pl.kernel(
    _body,
    out_type=jax.ShapeDtypeStruct((B, L, N, H), jnp.float32),
    mesh=plsc.VectorSubcoreMesh(core_axis_name="c", subcore_axis_name="s"),
    compiler_params=pltpu.CompilerParams(needs_layout_passes=False),
    scratch_types=[
        pltpu.VMEM((L * N,), jnp.int32),
        pltpu.VMEM((CL, H), jnp.float32),
        pltpu.VMEM((CL, H), jnp.float32),
        pltpu.VMEM((CL, H), jnp.float32),
        pltpu.VMEM((CL, H), jnp.float32),
        pltpu.VMEM((PCH, H), jnp.float32),
        pltpu.VMEM((CL, H), jnp.float32),
        pltpu.VMEM((H,), jnp.float32),
        pltpu.VMEM((H,), jnp.float32),
        pltpu.VMEM((544,), jnp.float32),
        pltpu.SemaphoreType.DMA,
        pltpu.SemaphoreType.DMA,
        pltpu.SemaphoreType.DMA,
        pltpu.SemaphoreType.DMA,
    ],
)


@jax.jit
def kernel(input_ids, word_table, pos_table, seg_table, gamma, beta):
    ids = jnp.concatenate(
        [jnp.full((B, N, 1), CLS_SEG, dtype=input_ids.dtype), input_ids],
        axis=2)
    idsT = ids.astype(jnp.int32).transpose(0, 2, 1).reshape(-1)
    posp = jnp.pad(pos_table, ((0, 7), (0, 0)))
    out = _sc_call(idsT, word_table, posp, seg_table, gamma, beta)
    return out.transpose(0, 2, 1, 3)


# speedup vs baseline: 1.7939x; 1.7939x over previous
import jax
import jax.numpy as jnp
from jax import lax
from jax.experimental import pallas as pl
from jax.experimental.pallas import tpu as pltpu
from jax.experimental.pallas import tpu_sc as plsc

B, N, K, H = 8, 64, 128, 768
V = 100000
L = K + 1
CLS_SEG = 1
EPS = 1e-12

NC, NS = 2, 16
NW = NC * NS
CL = 16
WPB = N // CL
NSL = H // 16
PCH = 16


def _rsqrt(x):
    xi = lax.bitcast_convert_type(x, jnp.int32)
    yi = jnp.full((16,), 0x5F3759DF, jnp.int32) - (xi >> 1)
    y = lax.bitcast_convert_type(yi, jnp.float32)
    for _ in range(3):
        y = y * (1.5 - 0.5 * x * y * y)
    return y


def _unit_norm(src, dst, seg_buf, pos_buf, pj, gam_buf, bet_buf, stats):
    lanes = lax.iota(jnp.int32, 16)

    def j_body(j, acc):
        sl = pl.ds(pl.multiple_of(j * 16, 16), 16)
        pos_j = pos_buf[pj, sl]
        out = []
        for i in range(16):
            x = src[i, sl] + seg_buf[i, sl] + pos_j
            dst[i, sl] = x
            out.append(acc[2 * i] + x)
            out.append(acc[2 * i + 1] + x * x)
        return tuple(out)

    acc0 = tuple(jnp.zeros((16,), jnp.float32) for _ in range(32))
    acc = lax.fori_loop(0, NSL, j_body, acc0)
    for i in range(16):
        plsc.store_scatter(stats, [lanes + 17 * i], acc[2 * i])
        plsc.store_scatter(stats, [lanes + (272 + 17 * i)], acc[2 * i + 1])
    tot_s = jnp.zeros((16,), jnp.float32)
    tot_q = jnp.zeros((16,), jnp.float32)
    for c in range(16):
        tot_s = tot_s + plsc.load_gather(stats, [lanes * 17 + c])
        tot_q = tot_q + plsc.load_gather(stats, [lanes * 17 + (272 + c)])
    mean = tot_s * (1.0 / H)
    msq = tot_q * (1.0 / H)
    rinv = _rsqrt(msq - mean * mean + EPS)
    shift = -mean * rinv

    for h in (0, 8):
        ri = [jnp.full((16,), rinv[i], jnp.float32) for i in range(h, h + 8)]
        sh = [jnp.full((16,), shift[i], jnp.float32) for i in range(h, h + 8)]

        def j2_body(j, carry, h=h, ri=ri, sh=sh):
            sl = pl.ds(pl.multiple_of(j * 16, 16), 16)
            g = gam_buf[sl]
            b = bet_buf[sl]
            for i in range(h, h + 8):
                dst[i, sl] = (dst[i, sl] * ri[i - h] + sh[i - h]) * g + b
            return carry

        lax.fori_loop(0, NSL, j2_body, 0)


def _body(ids_ref, word_ref, pos_ref, seg_ref, gamma_ref, beta_ref,
          out_ref,
          ids_buf, gbuf0, gbuf1, gbuf2, obuf0, obuf1, obuf2,
          pos_buf, seg_buf, gam_buf, bet_buf, stats,
          gsem0, gsem1, gsem2, ssem0, ssem1, ssem2):
    w = lax.axis_index("s") * NC + lax.axis_index("c")
    bb = w // WPB
    n0 = (w % WPB) * CL
    gbuf = (gbuf0, gbuf1, gbuf2)
    obuf = (obuf0, obuf1, obuf2)
    gsem = (gsem0, gsem1, gsem2)
    ssem = (ssem0, ssem1, ssem2)

    pltpu.sync_copy(gamma_ref, gam_buf)
    pltpu.sync_copy(beta_ref, bet_buf)
    pltpu.sync_copy(seg_ref.at[pl.ds(n0, CL)], seg_buf)
    pltpu.sync_copy(ids_ref.at[pl.ds(bb * (L * N), L * N)], ids_buf)

    def _idx(u):
        return ids_buf.at[pl.ds(pl.multiple_of(u * N + n0, 8), CL)]

    for s in range(3):
        pltpu.async_copy(word_ref.at[_idx(s)], gbuf[s], gsem[s])
        pltpu.async_copy(seg_buf, out_ref.at[bb, s, pl.ds(n0, CL)], ssem[s])

    def step(k, carry):
        for s in range(3):
            u = k * 3 + s
            pltpu.make_async_copy(
                pos_ref.at[pl.ds(0, CL)], obuf[s], ssem[s]).wait()
            pltpu.make_async_copy(
                word_ref.at[_idx(u)], gbuf[s], gsem[s]).wait()

            @pl.when(lax.rem(u, PCH) == 0)
            def _():
                lo = pl.multiple_of(u, PCH)
                pltpu.sync_copy(pos_ref.at[pl.ds(lo, PCH)], pos_buf)

            _unit_norm(gbuf[s], obuf[s], seg_buf, pos_buf, lax.rem(u, PCH),
                       gam_buf, bet_buf, stats)

            @pl.when(u + 3 < L)
            def _():
                pltpu.async_copy(word_ref.at[_idx(u + 3)], gbuf[s], gsem[s])

            pltpu.async_copy(obuf[s], out_ref.at[bb, u, pl.ds(n0, CL)],
                             ssem[s])
        return carry

    lax.fori_loop(0, L // 3, step, 0)

    for s in range(3):
        pltpu.make_async_copy(pos_ref.at[pl.ds(0, CL)], obuf[s],
                              ssem[s]).wait()


_sc_call = pl.kernel(
    _body,
    out_type=jax.ShapeDtypeStruct((B, L, N, H), jnp.float32),
    mesh=plsc.VectorSubcoreMesh(core_axis_name="c", subcore_axis_name="s"),
    compiler_params=pltpu.CompilerParams(needs_layout_passes=False),
    scratch_types=[
        pltpu.VMEM((L * N,), jnp.int32),
        pltpu.VMEM((CL, H), jnp.float32),
        pltpu.VMEM((CL, H), jnp.float32),
        pltpu.VMEM((CL, H), jnp.float32),
        pltpu.VMEM((CL, H), jnp.float32),
        pltpu.VMEM((CL, H), jnp.float32),
        pltpu.VMEM((CL, H), jnp.float32),
        pltpu.VMEM((PCH, H), jnp.float32),
        pltpu.VMEM((CL, H), jnp.float32),
        pltpu.VMEM((H,), jnp.float32),
        pltpu.VMEM((H,), jnp.float32),
        pltpu.VMEM((544,), jnp.float32),
        pltpu.SemaphoreType.DMA,
        pltpu.SemaphoreType.DMA,
        pltpu.SemaphoreType.DMA,
        pltpu.SemaphoreType.DMA,
        pltpu.SemaphoreType.DMA,
        pltpu.SemaphoreType.DMA,
    ],
)


@jax.jit
def kernel(input_ids, word_table, pos_table, seg_table, gamma, beta):
    ids = jnp.concatenate(
        [jnp.full((B, N, 1), CLS_SEG, dtype=input_ids.dtype), input_ids],
        axis=2)
    idsT = ids.astype(jnp.int32).transpose(0, 2, 1).reshape(-1)
    posp = jnp.pad(pos_table, ((0, 15), (0, 0)))
    out = _sc_call(idsT, word_table, posp, seg_table, gamma, beta)
    return out.transpose(0, 2, 1, 3)


# speedup vs baseline: 1.8293x; 1.0197x over previous
import jax
import jax.numpy as jnp
from jax import lax
from jax.experimental import pallas as pl
from jax.experimental.pallas import tpu as pltpu
from jax.experimental.pallas import tpu_sc as plsc

B, N, K, H = 8, 64, 128, 768
V = 100000
L = K + 1
CLS_SEG = 1
EPS = 1e-12

NC, NS = 2, 16
NW = NC * NS
CL = 16
WPB = N // CL
NSL = H // 16
PCH = 16


def _rsqrt(x):
    xi = lax.bitcast_convert_type(x, jnp.int32)
    yi = jnp.full((16,), 0x5F3759DF, jnp.int32) - (xi >> 1)
    y = lax.bitcast_convert_type(yi, jnp.float32)
    for _ in range(3):
        y = y * (1.5 - 0.5 * x * y * y)
    return y


def _unit_norm(src, dst, seg_buf, pos_buf, pj, gam_buf, bet_buf, stats):
    lanes = lax.iota(jnp.int32, 16)

    def j_body(j, acc):
        sl = pl.ds(pl.multiple_of(j * 16, 16), 16)
        pos_j = pos_buf[pj, sl]
        out = []
        for i in range(16):
            x = src[i, sl] + seg_buf[i, sl] + pos_j
            dst[i, sl] = x
            out.append(acc[2 * i] + x)
            out.append(acc[2 * i + 1] + x * x)
        return tuple(out)

    acc0 = tuple(jnp.zeros((16,), jnp.float32) for _ in range(32))
    acc = lax.fori_loop(0, NSL, j_body, acc0)
    for i in range(16):
        plsc.store_scatter(stats, [lanes + 17 * i], acc[2 * i])
        plsc.store_scatter(stats, [lanes + (272 + 17 * i)], acc[2 * i + 1])
    tot_s = jnp.zeros((16,), jnp.float32)
    tot_q = jnp.zeros((16,), jnp.float32)
    for c in range(16):
        tot_s = tot_s + plsc.load_gather(stats, [lanes * 17 + c])
        tot_q = tot_q + plsc.load_gather(stats, [lanes * 17 + (272 + c)])
    mean = tot_s * (1.0 / H)
    msq = tot_q * (1.0 / H)
    rinv = _rsqrt(msq - mean * mean + EPS)
    shift = -mean * rinv

    for h in (0, 8):
        ri = [jnp.full((16,), rinv[i], jnp.float32) for i in range(h, h + 8)]
        sh = [jnp.full((16,), shift[i], jnp.float32) for i in range(h, h + 8)]

        def j2_body(j, carry, h=h, ri=ri, sh=sh):
            sl = pl.ds(pl.multiple_of(j * 16, 16), 16)
            g = gam_buf[sl]
            b = bet_buf[sl]
            for i in range(h, h + 8):
                dst[i, sl] = (dst[i, sl] * ri[i - h] + sh[i - h]) * g + b
            return carry

        lax.fori_loop(0, NSL, j2_body, 0)


def _body(ids_ref, word_ref, pos_ref, seg_ref, gamma_ref, beta_ref,
          out_ref,
          ids_buf, gbuf0, gbuf1, obuf0, obuf1,
          pos_buf, seg_buf, gam_buf, bet_buf, stats,
          gsem0, gsem1, ssem0, ssem1):
    w = lax.axis_index("s") * NC + lax.axis_index("c")
    bb = w // WPB
    n0 = (w % WPB) * CL
    gbuf = (gbuf0, gbuf1)
    obuf = (obuf0, obuf1)
    gsem = (gsem0, gsem1)
    ssem = (ssem0, ssem1)

    pltpu.sync_copy(gamma_ref, gam_buf)
    pltpu.sync_copy(beta_ref, bet_buf)
    pltpu.sync_copy(seg_ref.at[pl.ds(n0, CL)], seg_buf)
    pltpu.sync_copy(ids_ref.at[pl.ds(bb * (L * N), L * N)], ids_buf)

    def _idx(u):
        return ids_buf.at[pl.ds(pl.multiple_of(u * N + n0, 8), CL)]

    for s in range(2):
        pltpu.async_copy(word_ref.at[_idx(s)], gbuf[s], gsem[s])
        pltpu.async_copy(seg_buf, out_ref.at[bb, s, pl.ds(n0, CL)], ssem[s])

    def step(k, carry):
        for s in range(2):
            u = k * 2 + s
            pltpu.make_async_copy(
                pos_ref.at[pl.ds(0, CL)], obuf[s], ssem[s]).wait()
            pltpu.make_async_copy(
                word_ref.at[_idx(u)], gbuf[s], gsem[s]).wait()

            if s == 0:
                @pl.when(lax.rem(u, PCH) == 0)
                def _():
                    lo = pl.multiple_of(u, PCH)
                    pltpu.sync_copy(pos_ref.at[pl.ds(lo, PCH)], pos_buf)

            _unit_norm(gbuf[s], obuf[s], seg_buf, pos_buf, lax.rem(u, PCH),
                       gam_buf, bet_buf, stats)

            @pl.when(u + 2 < L)
            def _():
                pltpu.async_copy(word_ref.at[_idx(u + 2)], gbuf[s], gsem[s])

            pltpu.async_copy(obuf[s], out_ref.at[bb, u, pl.ds(n0, CL)],
                             ssem[s])
        return carry

    lax.fori_loop(0, K // 2, step, 0)

    pltpu.make_async_copy(pos_ref.at[pl.ds(0, CL)], obuf0, ssem0).wait()
    pltpu.make_async_copy(word_ref.at[_idx(K)], gbuf0, gsem0).wait()
    pltpu.sync_copy(pos_ref.at[pl.ds(K, 8)], pos_buf.at[pl.ds(0, 8)])
    _unit_norm(gbuf0, obuf0, seg_buf, pos_buf, 0, gam_buf, bet_buf, stats)
    pltpu.sync_copy(obuf0, out_ref.at[bb, K, pl.ds(n0, CL)])
    pltpu.make_async_copy(pos_ref.at[pl.ds(0, CL)], obuf1, ssem1).wait()


_sc_call = pl.kernel(
    _body,
    out_type=jax.ShapeDtypeStruct((B, L, N, H), jnp.float32),
    mesh=plsc.VectorSubcoreMesh(core_axis_name="c", subcore_axis_name="s"),
    compiler_params=pltpu.CompilerParams(needs_layout_passes=False),
    scratch_types=[
        pltpu.VMEM((L * N,), jnp.int32),
        pltpu.VMEM((CL, H), jnp.float32),
        pltpu.VMEM((CL, H), jnp.float32),
        pltpu.VMEM((CL, H), jnp.float32),
        pltpu.VMEM((CL, H), jnp.float32),
        pltpu.VMEM((PCH, H), jnp.float32),
        pltpu.VMEM((CL, H), jnp.float32),
        pltpu.VMEM((H,), jnp.float32),
        pltpu.VMEM((H,), jnp.float32),
        pltpu.VMEM((544,), jnp.float32),
        pltpu.SemaphoreType.DMA,
        pltpu.SemaphoreType.DMA,
        pltpu.SemaphoreType.DMA,
        pltpu.SemaphoreType.DMA,
    ],
)


@jax.jit
def kernel(input_ids, word_table, pos_table, seg_table, gamma, beta):
    ids = jnp.concatenate(
        [jnp.full((B, N, 1), CLS_SEG, dtype=input_ids.dtype), input_ids],
        axis=2)
    idsT = ids.astype(jnp.int32).transpose(0, 2, 1).reshape(-1)
    posp = jnp.pad(pos_table, ((0, 7), (0, 0)))
    out = _sc_call(idsT, word_table, posp, seg_table, gamma, beta)
    return out.transpose(0, 2, 1, 3)
